# baseline (speedup 1.0000x reference)
import jax
import jax.numpy as jnp
from jax import lax
from jax.experimental import pallas as pl
from jax.experimental.pallas import tpu as pltpu

_H = 64
_V = 64
_B = 128
_L = 2048
_K = 4


def _body(seq_ref, embed_ref, W1_ref, b1_ref, W2_ref, b2_ref, gamma_ref,
          beta_ref, Wg1_ref, bg1_ref, Wg2_ref, bg2_ref, Wq_ref, bq_ref,
          Wout_ref, bout_ref, out_ref):
    f32 = jnp.float32

    E = embed_ref[...]
    h1 = jnp.maximum(
        jnp.dot(E, W1_ref[...], preferred_element_type=f32) + b1_ref[...], 0.0)
    ff = jnp.dot(h1, W2_ref[...], preferred_element_type=f32) + b2_ref[...]
    X = E + ff
    mu = jnp.mean(X, axis=1, keepdims=True)
    var = jnp.mean((X - mu) ** 2, axis=1, keepdims=True)
    Hv = (X - mu) / jnp.sqrt(var + 1e-5) * gamma_ref[...] + beta_ref[...]

    g1 = jnp.maximum(
        jnp.dot(Hv, Wg1_ref[...], preferred_element_type=f32) + bg1_ref[...], 0.0)
    gl = jnp.dot(g1, Wg2_ref[...], preferred_element_type=f32) + bg2_ref[...]

    iu = lax.broadcasted_iota(jnp.int32, (_V, _V), 0)
    it = lax.broadcasted_iota(jnp.int32, (_V, _V), 1)
    gcol = jnp.broadcast_to(gl, (_V, _V))
    grow = jnp.sum(jnp.where(iu == it, gcol, 0.0), axis=0, keepdims=True)
    ahead = ((gcol > grow) | ((gcol == grow) & (iu < it))).astype(f32)

    seq_all = seq_ref[...]
    lane_v = lax.broadcasted_iota(jnp.int32, (_B, _V), 1)

    def hist(v, acc):
        eq = (seq_all == v).astype(f32)
        cnt = jnp.sum(eq, axis=1, keepdims=True)
        return jnp.where(lane_v == v, cnt, acc)

    counts = jax.lax.fori_loop(0, _V, hist,
                               jnp.zeros((_B, _V), f32), unroll=4)

    A = lax.dot_general(counts, ahead, (((1,), (0,)), ((), ())),
                        preferred_element_type=f32)
    m_tok = jnp.minimum(counts, jnp.maximum(float(_K) - A, 0.0))

    lt = seq_all[:, _L - 1:_L]
    OL = (jnp.broadcast_to(lt, (_B, _V)) == lane_v).astype(f32)
    qh = jnp.dot(OL, Hv, preferred_element_type=f32)
    q = jnp.dot(qh, Wq_ref[...], preferred_element_type=f32) + bq_ref[...]

    S = lax.dot_general(q, Hv, (((1,), (1,)), ((), ())),
                        preferred_element_type=f32) * 0.125
    sel = m_tok > 0.0
    smax = jnp.max(jnp.where(sel, S, -1e30), axis=1, keepdims=True)
    w = m_tok * jnp.exp(jnp.where(sel, S - smax, 0.0))
    Z = jnp.sum(w, axis=1, keepdims=True)
    wn = w / Z
    pooled = jnp.dot(wn, Hv, preferred_element_type=f32)
    out_ref[...] = (jnp.dot(pooled, Wout_ref[...], preferred_element_type=f32)
                    + bout_ref[...])


def _prep(seq, embed, W1, b1, W2, b2, gamma, beta, Wg1, bg1, Wg2, bg2,
          Wq, bq, Wout, bout):
    r = lambda x: x.reshape(1, -1)
    return (seq, embed, W1, r(b1), W2, r(b2), r(gamma), r(beta),
            Wg1, r(bg1), Wg2, r(bg2), Wq, r(bq), Wout, r(bout))


def kernel(seq, embed, W1, b1, W2, b2, gamma, beta, Wg1, bg1, Wg2, bg2,
           Wq, bq, Wout, bout):
    args = _prep(seq, embed, W1, b1, W2, b2, gamma, beta, Wg1, bg1, Wg2, bg2,
                 Wq, bq, Wout, bout)
    return pl.pallas_call(
        _body,
        out_shape=jax.ShapeDtypeStruct((_B, _V), jnp.float32),
    )(*args)


# speedup vs baseline: 1.1983x; 1.1983x over previous
import jax
import jax.numpy as jnp
from jax import lax
from jax.experimental import pallas as pl
from jax.experimental.pallas import tpu as pltpu

_H = 64
_V = 64
_B = 128
_L = 2048
_K = 4


def _body(seq_ref, embed_ref, W1_ref, b1_ref, W2_ref, b2_ref, gamma_ref,
          beta_ref, Wg1_ref, bg1_ref, Wg2_ref, bg2_ref, Wq_ref, bq_ref,
          Wout_ref, bout_ref, out_ref):
    f32 = jnp.float32

    E = embed_ref[...]
    h1 = jnp.maximum(
        jnp.dot(E, W1_ref[...], preferred_element_type=f32) + b1_ref[...], 0.0)
    ff = jnp.dot(h1, W2_ref[...], preferred_element_type=f32) + b2_ref[...]
    X = E + ff
    mu = jnp.mean(X, axis=1, keepdims=True)
    var = jnp.mean((X - mu) ** 2, axis=1, keepdims=True)
    Hv = (X - mu) / jnp.sqrt(var + 1e-5) * gamma_ref[...] + beta_ref[...]

    g1 = jnp.maximum(
        jnp.dot(Hv, Wg1_ref[...], preferred_element_type=f32) + bg1_ref[...], 0.0)
    gl = jnp.dot(g1, Wg2_ref[...], preferred_element_type=f32) + bg2_ref[...]

    iu = lax.broadcasted_iota(jnp.int32, (_V, _V), 0)
    it = lax.broadcasted_iota(jnp.int32, (_V, _V), 1)
    gcol = jnp.broadcast_to(gl, (_V, _V))
    grow = jnp.sum(jnp.where(iu == it, gcol, 0.0), axis=0, keepdims=True)
    ahead = ((gcol > grow) | ((gcol == grow) & (iu < it))).astype(f32)

    seq_all = seq_ref[...]
    bf16 = jnp.bfloat16
    seqb = seq_all.astype(bf16)
    lane_v = lax.broadcasted_iota(jnp.int32, (_B, _V), 1)
    oneb = jnp.ones((_B, _L), bf16)
    zerob = jnp.zeros((_B, _L), bf16)

    def hist(v, acc):
        eq = seqb == v.astype(bf16)
        x = jnp.where(eq, oneb, zerob)
        x = x[:, :1024] + x[:, 1024:]
        x = x[:, :512] + x[:, 512:]
        x = x[:, :256] + x[:, 256:]
        cnt = jnp.sum(x.astype(f32), axis=1, keepdims=True)
        return jnp.where(lane_v == v, cnt, acc)

    counts = jax.lax.fori_loop(0, _V, hist,
                               jnp.zeros((_B, _V), f32), unroll=4)

    A = lax.dot_general(counts, ahead, (((1,), (0,)), ((), ())),
                        preferred_element_type=f32)
    m_tok = jnp.minimum(counts, jnp.maximum(float(_K) - A, 0.0))

    lt = seq_all[:, _L - 1:_L]
    OL = (jnp.broadcast_to(lt, (_B, _V)) == lane_v).astype(f32)
    qh = jnp.dot(OL, Hv, preferred_element_type=f32)
    q = jnp.dot(qh, Wq_ref[...], preferred_element_type=f32) + bq_ref[...]

    S = lax.dot_general(q, Hv, (((1,), (1,)), ((), ())),
                        preferred_element_type=f32) * 0.125
    sel = m_tok > 0.0
    smax = jnp.max(jnp.where(sel, S, -1e30), axis=1, keepdims=True)
    w = m_tok * jnp.exp(jnp.where(sel, S - smax, 0.0))
    Z = jnp.sum(w, axis=1, keepdims=True)
    wn = w / Z
    pooled = jnp.dot(wn, Hv, preferred_element_type=f32)
    out_ref[...] = (jnp.dot(pooled, Wout_ref[...], preferred_element_type=f32)
                    + bout_ref[...])


def _prep(seq, embed, W1, b1, W2, b2, gamma, beta, Wg1, bg1, Wg2, bg2,
          Wq, bq, Wout, bout):
    r = lambda x: x.reshape(1, -1)
    return (seq, embed, W1, r(b1), W2, r(b2), r(gamma), r(beta),
            Wg1, r(bg1), Wg2, r(bg2), Wq, r(bq), Wout, r(bout))


def kernel(seq, embed, W1, b1, W2, b2, gamma, beta, Wg1, bg1, Wg2, bg2,
           Wq, bq, Wout, bout):
    args = _prep(seq, embed, W1, b1, W2, b2, gamma, beta, Wg1, bg1, Wg2, bg2,
                 Wq, bq, Wout, bout)
    return pl.pallas_call(
        _body,
        out_shape=jax.ShapeDtypeStruct((_B, _V), jnp.float32),
    )(*args)


# speedup vs baseline: 1.2691x; 1.0592x over previous
import jax
import jax.numpy as jnp
from jax import lax
from jax.experimental import pallas as pl
from jax.experimental.pallas import tpu as pltpu

_H = 64
_V = 64
_B = 128
_L = 2048
_K = 4


def _body(seq_ref, embed_ref, W1_ref, b1_ref, W2_ref, b2_ref, gamma_ref,
          beta_ref, Wg1_ref, bg1_ref, Wg2_ref, bg2_ref, Wq_ref, bq_ref,
          Wout_ref, bout_ref, out_ref):
    f32 = jnp.float32

    E = embed_ref[...]
    h1 = jnp.maximum(
        jnp.dot(E, W1_ref[...], preferred_element_type=f32) + b1_ref[...], 0.0)
    ff = jnp.dot(h1, W2_ref[...], preferred_element_type=f32) + b2_ref[...]
    X = E + ff
    mu = jnp.mean(X, axis=1, keepdims=True)
    var = jnp.mean((X - mu) ** 2, axis=1, keepdims=True)
    Hv = (X - mu) / jnp.sqrt(var + 1e-5) * gamma_ref[...] + beta_ref[...]

    g1 = jnp.maximum(
        jnp.dot(Hv, Wg1_ref[...], preferred_element_type=f32) + bg1_ref[...], 0.0)
    gl = jnp.dot(g1, Wg2_ref[...], preferred_element_type=f32) + bg2_ref[...]

    iu = lax.broadcasted_iota(jnp.int32, (_V, _V), 0)
    it = lax.broadcasted_iota(jnp.int32, (_V, _V), 1)
    gcol = jnp.broadcast_to(gl, (_V, _V))
    grow = jnp.sum(jnp.where(iu == it, gcol, 0.0), axis=0, keepdims=True)
    ahead = ((gcol > grow) | ((gcol == grow) & (iu < it))).astype(f32)

    seq_all = seq_ref[...]
    bf16 = jnp.bfloat16
    seqb = seq_all.astype(bf16)
    lane_v = lax.broadcasted_iota(jnp.int32, (_B, _V), 1)
    oneb = jnp.ones((_B, _L), bf16)
    zerob = jnp.zeros((_B, _L), bf16)

    def hist(v, acc):
        eq = seqb == v.astype(bf16)
        x = jnp.where(eq, oneb, zerob)
        x = x[:, :1024] + x[:, 1024:]
        x = x[:, :512] + x[:, 512:]
        x = x[:, :256] + x[:, 256:]
        cnt = jnp.sum(x.astype(f32), axis=1, keepdims=True)
        return jnp.where(lane_v == v, cnt, acc)

    counts = jax.lax.fori_loop(0, _V, hist,
                               jnp.zeros((_B, _V), f32), unroll=8)

    A = lax.dot_general(counts, ahead, (((1,), (0,)), ((), ())),
                        preferred_element_type=f32)
    m_tok = jnp.minimum(counts, jnp.maximum(float(_K) - A, 0.0))

    lt = seq_all[:, _L - 1:_L]
    OL = (jnp.broadcast_to(lt, (_B, _V)) == lane_v).astype(f32)
    qh = jnp.dot(OL, Hv, preferred_element_type=f32)
    q = jnp.dot(qh, Wq_ref[...], preferred_element_type=f32) + bq_ref[...]

    S = lax.dot_general(q, Hv, (((1,), (1,)), ((), ())),
                        preferred_element_type=f32) * 0.125
    sel = m_tok > 0.0
    smax = jnp.max(jnp.where(sel, S, -1e30), axis=1, keepdims=True)
    w = m_tok * jnp.exp(jnp.where(sel, S - smax, 0.0))
    Z = jnp.sum(w, axis=1, keepdims=True)
    wn = w / Z
    pooled = jnp.dot(wn, Hv, preferred_element_type=f32)
    out_ref[...] = (jnp.dot(pooled, Wout_ref[...], preferred_element_type=f32)
                    + bout_ref[...])


def _prep(seq, embed, W1, b1, W2, b2, gamma, beta, Wg1, bg1, Wg2, bg2,
          Wq, bq, Wout, bout):
    r = lambda x: x.reshape(1, -1)
    return (seq, embed, W1, r(b1), W2, r(b2), r(gamma), r(beta),
            Wg1, r(bg1), Wg2, r(bg2), Wq, r(bq), Wout, r(bout))


def kernel(seq, embed, W1, b1, W2, b2, gamma, beta, Wg1, bg1, Wg2, bg2,
           Wq, bq, Wout, bout):
    args = _prep(seq, embed, W1, b1, W2, b2, gamma, beta, Wg1, bg1, Wg2, bg2,
                 Wq, bq, Wout, bout)
    return pl.pallas_call(
        _body,
        out_shape=jax.ShapeDtypeStruct((_B, _V), jnp.float32),
    )(*args)
